# baseline (speedup 1.0000x reference)
import functools

import jax
import jax.numpy as jnp
from jax import lax
from jax.experimental import pallas as pl
from jax.experimental.pallas import tpu as pltpu
from jax.experimental.pallas import tpu_sc as plsc

B = 1024
CTX = 50
EMB = 64
VOC = 100000

NC = 2
NS = 16
NW = NC * NS
N_IDX = B * CTX
B_PER_W = B // NW
IDX_PER_W = B_PER_W * CTX
MAX_DESC = 128
LANES = 16
EMB_VREGS = EMB // LANES


ROW128 = 2 * EMB
E_PER_CHUNK = 8
CHUNK = E_PER_CHUNK * CTX
N_CHUNKS = B_PER_W // E_PER_CHUNK


def _pool_body(idx_hbm, table_hbm, out_hbm, idx_v, buf, pooled_v, sem):
    wid = lax.axis_index("s") * NC + lax.axis_index("c")
    base = wid * IDX_PER_W
    ebase = wid * B_PER_W
    pltpu.sync_copy(idx_hbm.at[pl.ds(base, IDX_PER_W)], idx_v)

    inv_ctx = jnp.float32(1.0 / CTX)

    def chunk_body(c, carry):
        p = c % 2
        descs = []
        off = 0
        while off < CHUNK:
            n = min(MAX_DESC, CHUNK - off)
            descs.append(
                pltpu.async_copy(
                    table_hbm.at[idx_v.at[pl.ds(c * CHUNK + off, n)]],
                    buf.at[p, pl.ds(off, n), :],
                    sem,
                )
            )
            off += n
        for d in descs:
            d.wait()
        for el in range(E_PER_CHUNK):
            acc = [jnp.zeros((LANES,), jnp.float32)] * EMB_VREGS
            for r in range(CTX):
                row = el * CTX + r
                for j in range(EMB_VREGS):
                    acc[j] = acc[j] + buf[p, row, pl.ds(j * LANES, LANES)]
            for j in range(EMB_VREGS):
                pooled_v[el, pl.ds(j * LANES, LANES)] = acc[j] * inv_ctx
        pltpu.sync_copy(
            pooled_v, out_hbm.at[pl.ds(ebase + c * E_PER_CHUNK, E_PER_CHUNK)]
        )
        return carry

    lax.fori_loop(0, N_CHUNKS, chunk_body, 0)


_pool_sc = functools.partial(
    pl.kernel,
    out_type=jax.ShapeDtypeStruct((B, EMB), jnp.float32),
    mesh=plsc.VectorSubcoreMesh(
        core_axis_name="c", subcore_axis_name="s", num_cores=NC,
        num_subcores=NS,
    ),
    scratch_types=[
        pltpu.VMEM((IDX_PER_W,), jnp.int32),
        pltpu.VMEM((2, CHUNK, ROW128), jnp.float32),
        pltpu.VMEM((E_PER_CHUNK, EMB), jnp.float32),
        pltpu.SemaphoreType.DMA,
    ],
)(_pool_body)


V_BLK = 2048


def _tr_body(t_ref, o_ref):
    eye = (
        lax.broadcasted_iota(jnp.int32, (EMB, EMB), 0)
        == lax.broadcasted_iota(jnp.int32, (EMB, EMB), 1)
    ).astype(jnp.float32)
    o_ref[:, :EMB] = lax.dot_general(
        t_ref[...],
        eye,
        (((0,), (0,)), ((), ())),
        preferred_element_type=jnp.float32,
    )


def _format_tc(tableT):
    return pl.pallas_call(
        _tr_body,
        grid=(pl.cdiv(VOC, V_BLK),),
        in_specs=[pl.BlockSpec((EMB, V_BLK), lambda i: (0, i))],
        out_specs=pl.BlockSpec((V_BLK, ROW128), lambda i: (i, 0)),
        out_shape=jax.ShapeDtypeStruct((VOC, ROW128), jnp.float32),
        compiler_params=pltpu.CompilerParams(
            dimension_semantics=("arbitrary",),
        ),
    )(tableT)


N_BLK = 4096


def _mm_body(w_ref, p_ref, b_ref, o_ref):
    wb = w_ref[...].astype(jnp.bfloat16)
    pb = p_ref[...].astype(jnp.bfloat16)
    bias8 = jnp.broadcast_to(b_ref[...] * 0.125, (8, N_BLK)).astype(
        jnp.bfloat16
    )
    ones8 = jnp.ones((B, 8), jnp.bfloat16)
    o_ref[...] = lax.dot_general(
        jnp.concatenate([wb, bias8], axis=0),
        jnp.concatenate([pb, ones8], axis=1),
        (((0,), (1,)), ((), ())),
        preferred_element_type=jnp.float32,
    )


def _matmul_tc(pooled, W, brow):
    n_blocks = pl.cdiv(VOC, N_BLK)
    return pl.pallas_call(
        _mm_body,
        grid=(n_blocks,),
        in_specs=[
            pl.BlockSpec((EMB, N_BLK), lambda i: (0, i)),
            pl.BlockSpec((B, EMB), lambda i: (0, 0)),
            pl.BlockSpec((1, N_BLK), lambda i: (0, i)),
        ],
        out_specs=pl.BlockSpec((N_BLK, B), lambda i: (i, 0)),
        out_shape=jax.ShapeDtypeStruct((VOC, B), jnp.float32),
        compiler_params=pltpu.CompilerParams(
            dimension_semantics=("arbitrary",),
        ),
    )(W, pooled, brow)


@jax.jit
def kernel(inpt, table, W, b):
    idx_flat = inpt.astype(jnp.int32).reshape(N_IDX)
    table128 = _format_tc(table.T)
    pooled = _pool_sc(idx_flat, table128)
    outT = _matmul_tc(pooled, W, b.reshape(1, VOC))
    return outT.T


# speedup vs baseline: 1.0083x; 1.0083x over previous
import functools

import jax
import jax.numpy as jnp
from jax import lax
from jax.experimental import pallas as pl
from jax.experimental.pallas import tpu as pltpu
from jax.experimental.pallas import tpu_sc as plsc

B = 1024
CTX = 50
EMB = 64
VOC = 100000

NC = 2
NS = 16
NW = NC * NS
N_IDX = B * CTX
B_PER_W = B // NW
IDX_PER_W = B_PER_W * CTX
MAX_DESC = 128
LANES = 16
EMB_VREGS = EMB // LANES


ROW128 = 2 * EMB
E_PER_CHUNK = 8
CHUNK = E_PER_CHUNK * CTX
N_CHUNKS = B_PER_W // E_PER_CHUNK


def _pool_body(idx_hbm, table_hbm, out_hbm, idx_v, buf, pooled_v, sem):
    wid = lax.axis_index("s") * NC + lax.axis_index("c")
    base = wid * IDX_PER_W
    ebase = wid * B_PER_W
    pltpu.sync_copy(idx_hbm.at[pl.ds(base, IDX_PER_W)], idx_v)

    inv_ctx = jnp.float32(1.0 / CTX)

    def chunk_body(c, carry):
        p = c % 2
        descs = []
        off = 0
        while off < CHUNK:
            n = min(MAX_DESC, CHUNK - off)
            descs.append(
                pltpu.async_copy(
                    table_hbm.at[idx_v.at[pl.ds(c * CHUNK + off, n)]],
                    buf.at[p, pl.ds(off, n), :],
                    sem,
                )
            )
            off += n
        for d in descs:
            d.wait()
        for el in range(E_PER_CHUNK):
            acc = [jnp.zeros((LANES,), jnp.float32)] * EMB_VREGS
            for r in range(CTX):
                row = el * CTX + r
                for j in range(EMB_VREGS):
                    acc[j] = acc[j] + buf[p, row, pl.ds(j * LANES, LANES)]
            for j in range(EMB_VREGS):
                pooled_v[el, pl.ds(j * LANES, LANES)] = acc[j] * inv_ctx
        pltpu.sync_copy(
            pooled_v, out_hbm.at[pl.ds(ebase + c * E_PER_CHUNK, E_PER_CHUNK)]
        )
        return carry

    lax.fori_loop(0, N_CHUNKS, chunk_body, 0)


_pool_sc = functools.partial(
    pl.kernel,
    out_type=jax.ShapeDtypeStruct((B, EMB), jnp.float32),
    mesh=plsc.VectorSubcoreMesh(
        core_axis_name="c", subcore_axis_name="s", num_cores=NC,
        num_subcores=NS,
    ),
    scratch_types=[
        pltpu.VMEM((IDX_PER_W,), jnp.int32),
        pltpu.VMEM((2, CHUNK, ROW128), jnp.float32),
        pltpu.VMEM((E_PER_CHUNK, EMB), jnp.float32),
        pltpu.SemaphoreType.DMA,
    ],
)(_pool_body)


V_BLK = 2048


def _tr_body(t_ref, o_ref):
    o_ref[:, :EMB] = t_ref[...].T


def _format_tc(tableT):
    return pl.pallas_call(
        _tr_body,
        grid=(pl.cdiv(VOC, V_BLK),),
        in_specs=[pl.BlockSpec((EMB, V_BLK), lambda i: (0, i))],
        out_specs=pl.BlockSpec((V_BLK, ROW128), lambda i: (i, 0)),
        out_shape=jax.ShapeDtypeStruct((VOC, ROW128), jnp.float32),
        compiler_params=pltpu.CompilerParams(
            dimension_semantics=("arbitrary",),
        ),
    )(tableT)


N_BLK = 4096


def _mm_body(w_ref, p_ref, b_ref, o_ref):
    wb = w_ref[...].astype(jnp.bfloat16)
    pb = p_ref[...].astype(jnp.bfloat16)
    bias8 = jnp.broadcast_to(b_ref[...] * 0.125, (8, N_BLK)).astype(
        jnp.bfloat16
    )
    ones8 = jnp.ones((B, 8), jnp.bfloat16)
    o_ref[...] = lax.dot_general(
        jnp.concatenate([wb, bias8], axis=0),
        jnp.concatenate([pb, ones8], axis=1),
        (((0,), (1,)), ((), ())),
        preferred_element_type=jnp.float32,
    )


def _matmul_tc(pooled, W, brow):
    n_blocks = pl.cdiv(VOC, N_BLK)
    return pl.pallas_call(
        _mm_body,
        grid=(n_blocks,),
        in_specs=[
            pl.BlockSpec((EMB, N_BLK), lambda i: (0, i)),
            pl.BlockSpec((B, EMB), lambda i: (0, 0)),
            pl.BlockSpec((1, N_BLK), lambda i: (0, i)),
        ],
        out_specs=pl.BlockSpec((N_BLK, B), lambda i: (i, 0)),
        out_shape=jax.ShapeDtypeStruct((VOC, B), jnp.float32),
        compiler_params=pltpu.CompilerParams(
            dimension_semantics=("arbitrary",),
            fuse_transposed_lhs_in_matmul=True,
        ),
    )(W, pooled, brow)


@jax.jit
def kernel(inpt, table, W, b):
    idx_flat = inpt.astype(jnp.int32).reshape(N_IDX)
    table128 = _format_tc(table.T)
    pooled = _pool_sc(idx_flat, table128)
    outT = _matmul_tc(pooled, W, b.reshape(1, VOC))
    return outT.T
